# Initial kernel scaffold; baseline (speedup 1.0000x reference)
import functools
import jax
import jax.numpy as jnp
from jax import lax
from jax.experimental import pallas as pl
from jax.experimental.pallas import tpu as pltpu
from jax.experimental.pallas import tpu_sc as plsc

BB, NN, KK = 8, 1024, 20
LAT = 1024
SLOPE = 0.2
NWORK = 32


def _topk_body(x_ref, idx_ref):
    b = pl.program_id(0)
    xb = x_ref[0]
    sq = jnp.sum(xb * xb, axis=1, keepdims=True)
    xx = lax.dot_general(xb, xb, (((1,), (1,)), ((), ())),
                         preferred_element_type=jnp.float32)
    d2 = sq + jnp.transpose(sq) - 2.0 * xx
    d2 = jnp.maximum(d2, 0.0)
    iota = lax.broadcasted_iota(jnp.int32, (NN, NN), 1)
    kio = lax.broadcasted_iota(jnp.int32, (NN, KK), 1)
    acc = jnp.zeros((NN, KK), jnp.int32)
    for k in range(KK):
        m = jnp.min(d2, axis=1, keepdims=True)
        am = jnp.min(jnp.where(d2 == m, iota, NN), axis=1, keepdims=True)
        acc = jnp.where(kio == k, am, acc)
        d2 = jnp.where(iota == am, jnp.inf, d2)
    idx_ref[0] = acc + b * NN


def _topk(x):
    return pl.pallas_call(
        _topk_body,
        grid=(BB,),
        in_specs=[pl.BlockSpec((1, NN, 3), lambda b: (b, 0, 0))],
        out_specs=pl.BlockSpec((1, NN, KK), lambda b: (b, 0, 0)),
        out_shape=jax.ShapeDtypeStruct((BB, NN, KK), jnp.int32),
    )(x)


def _pq_body(f_ref, w_ref, p_ref, q_ref):
    f = f_ref[...]
    D = f.shape[1]
    wt = w_ref[0:D]
    wb = w_ref[D:]
    p_ref[...] = lax.dot_general(f, wt, (((1,), (0,)), ((), ())),
                                 preferred_element_type=jnp.float32)
    q_ref[...] = lax.dot_general(f, wb - wt, (((1,), (0,)), ((), ())),
                                 preferred_element_type=jnp.float32)


def _pq(feat, W):
    C = W.shape[1]
    return pl.pallas_call(
        _pq_body,
        out_shape=(jax.ShapeDtypeStruct((BB * NN, C), jnp.float32),
                   jax.ShapeDtypeStruct((BB * NN, C), jnp.float32)),
    )(feat, W)


def _make_gather_reduce(C):
    GPW = (BB * NN) // NWORK
    CH = 4
    NCH = GPW // CH
    IDXW = GPW * KK
    BN = BB * NN
    mesh = plsc.VectorSubcoreMesh(core_axis_name="c", subcore_axis_name="s")

    @functools.partial(
        pl.kernel,
        mesh=mesh,
        out_type=jax.ShapeDtypeStruct((3 * BN, C), jnp.float32),
        scratch_types=[
            pltpu.VMEM((IDXW,), jnp.int32),
            pltpu.VMEM((CH * KK, C), jnp.float32),
            pltpu.VMEM((CH, C), jnp.float32),
            pltpu.VMEM((CH, C), jnp.float32),
            pltpu.VMEM((CH, C), jnp.float32),
            pltpu.SemaphoreType.DMA,
        ],
    )
    def sc_kern(p_hbm, idx_hbm, o_hbm, idx_v, rows_v, m_v, s_v, q_v, sem):
        wid = lax.axis_index("s") * 2 + lax.axis_index("c")
        ibase = wid * IDXW
        pltpu.sync_copy(idx_hbm.at[pl.ds(ibase, IDXW)], idx_v)

        @pl.loop(0, NCH)
        def _chunk(ch):
            off = pl.multiple_of(ch * (CH * KK), 8)
            pltpu.async_copy(p_hbm.at[idx_v.at[pl.ds(off, CH * KK)]],
                             rows_v, sem).wait()

            @pl.loop(0, CH)
            def _group(g):
                base = g * KK
                for cb in range(C // 16):
                    sl = pl.ds(cb * 16, 16)
                    v0 = rows_v[base, sl]

                    def kbody(k, carry):
                        vm, vs, vq = carry
                        v = rows_v[base + k, sl]
                        return (jnp.maximum(vm, v), vs + v, vq + v * v)

                    vm, vs, vq = lax.fori_loop(1, KK, kbody,
                                               (v0, v0, v0 * v0))
                    m_v[g, sl] = vm
                    s_v[g, sl] = vs
                    q_v[g, sl] = vq

            row0 = wid * GPW + ch * CH
            pltpu.sync_copy(m_v, o_hbm.at[pl.ds(row0, CH)])
            pltpu.sync_copy(s_v, o_hbm.at[pl.ds(BN + row0, CH)])
            pltpu.sync_copy(q_v, o_hbm.at[pl.ds(2 * BN + row0, CH)])

    return sc_kern


_GR_CACHE = {}


def _gather_reduce(P, idx_flat):
    C = P.shape[1]
    if C not in _GR_CACHE:
        _GR_CACHE[C] = _make_gather_reduce(C)
    return _GR_CACHE[C](P, idx_flat)


def _combine_body(o_ref, q_ref, g_ref, b_ref, x_ref):
    BN = BB * NN
    M = o_ref[0:BN]
    S = o_ref[BN:2 * BN]
    S2 = o_ref[2 * BN:3 * BN]
    Q = q_ref[...]
    cnt = float(BN * KK)
    sum_y = (jnp.sum(S, axis=0, keepdims=True)
             + KK * jnp.sum(Q, axis=0, keepdims=True))
    sumsq = (jnp.sum(S2, axis=0, keepdims=True)
             + 2.0 * jnp.sum(Q * S, axis=0, keepdims=True)
             + KK * jnp.sum(Q * Q, axis=0, keepdims=True))
    mean = sum_y / cnt
    var = sumsq / cnt - mean * mean
    scale = g_ref[...] * lax.rsqrt(var + 1e-5)
    shift = b_ref[...] - mean * scale
    t = (M + Q) * scale + shift
    x_ref[...] = jnp.where(t >= 0, t, SLOPE * t)


def _combine(O, Q, g, b):
    C = Q.shape[1]
    return pl.pallas_call(
        _combine_body,
        out_shape=jax.ShapeDtypeStruct((BB * NN, C), jnp.float32),
    )(O, Q, g.reshape(1, C), b.reshape(1, C))


def _final_body(x1_ref, x2_ref, x3_ref, x4_ref, w_ref, g_ref, b_ref,
                out_ref, max_s, sum_s, sq_s):
    b = pl.program_id(0)
    x5 = lax.dot_general(x1_ref[...], w_ref[0:64],
                         (((1,), (0,)), ((), ())),
                         preferred_element_type=jnp.float32)
    x5 += lax.dot_general(x2_ref[...], w_ref[64:128],
                          (((1,), (0,)), ((), ())),
                          preferred_element_type=jnp.float32)
    x5 += lax.dot_general(x3_ref[...], w_ref[128:256],
                          (((1,), (0,)), ((), ())),
                          preferred_element_type=jnp.float32)
    x5 += lax.dot_general(x4_ref[...], w_ref[256:512],
                          (((1,), (0,)), ((), ())),
                          preferred_element_type=jnp.float32)
    bio = lax.broadcasted_iota(jnp.int32, (BB, LAT), 0)
    mx = jnp.max(x5, axis=0, keepdims=True)
    max_s[...] = jnp.where(bio == b, mx, max_s[...])
    sm = jnp.sum(x5, axis=0, keepdims=True)
    sq = jnp.sum(x5 * x5, axis=0, keepdims=True)

    @pl.when(b == 0)
    def _():
        sum_s[...] = sm
        sq_s[...] = sq

    @pl.when(b > 0)
    def _():
        sum_s[...] += sm
        sq_s[...] += sq

    @pl.when(b == BB - 1)
    def _():
        cnt = float(BB * NN)
        mean = sum_s[...] / cnt
        var = sq_s[...] / cnt - mean * mean
        scale = g_ref[...] * lax.rsqrt(var + 1e-5)
        shift = b_ref[...] - mean * scale
        t = max_s[...] * scale + shift
        out_ref[...] = jnp.where(t >= 0, t, SLOPE * t)


def _final(x1, x2, x3, x4, W5, g5, b5):
    return pl.pallas_call(
        _final_body,
        grid=(BB,),
        in_specs=[
            pl.BlockSpec((NN, 64), lambda b: (b, 0)),
            pl.BlockSpec((NN, 64), lambda b: (b, 0)),
            pl.BlockSpec((NN, 128), lambda b: (b, 0)),
            pl.BlockSpec((NN, 256), lambda b: (b, 0)),
            pl.BlockSpec((512, LAT), lambda b: (0, 0)),
            pl.BlockSpec((1, LAT), lambda b: (0, 0)),
            pl.BlockSpec((1, LAT), lambda b: (0, 0)),
        ],
        out_specs=pl.BlockSpec((BB, LAT), lambda b: (0, 0)),
        out_shape=jax.ShapeDtypeStruct((BB, LAT), jnp.float32),
        scratch_shapes=[
            pltpu.VMEM((BB, LAT), jnp.float32),
            pltpu.VMEM((1, LAT), jnp.float32),
            pltpu.VMEM((1, LAT), jnp.float32),
        ],
    )(x1, x2, x3, x4, W5, g5.reshape(1, LAT), b5.reshape(1, LAT))


def kernel(x, W1, W2, W3, W4, W5, g1, b1, g2, b2, g3, b3, g4, b4, g5, b5):
    idx = _topk(x)
    idx_flat = idx.reshape(BB * NN * KK)

    feat = x.reshape(BB * NN, 3)
    outs = []
    for W, g, b in ((W1, g1, b1), (W2, g2, b2), (W3, g3, b3), (W4, g4, b4)):
        P, Q = _pq(feat, W)
        O = _gather_reduce(P, idx_flat)
        feat = _combine(O, Q, g, b)
        outs.append(feat)

    return _final(outs[0], outs[1], outs[2], outs[3], W5, g5, b5)


# speedup vs baseline: 9.8874x; 9.8874x over previous
import functools
import jax
import jax.numpy as jnp
from jax import lax
from jax.experimental import pallas as pl
from jax.experimental.pallas import tpu as pltpu
from jax.experimental.pallas import tpu_sc as plsc

BB, NN, KK = 8, 1024, 20
LAT = 1024
SLOPE = 0.2
NWORK = 32


def _topk_body(x_ref, idx_ref):
    b = pl.program_id(0)
    xb = x_ref[0]
    sq = jnp.sum(xb * xb, axis=1, keepdims=True)
    xx = lax.dot_general(xb, xb, (((1,), (1,)), ((), ())),
                         preferred_element_type=jnp.float32)
    d2 = sq + jnp.transpose(sq) - 2.0 * xx
    d2 = jnp.maximum(d2, 0.0)
    iota = lax.broadcasted_iota(jnp.int32, (NN, NN), 1)
    kio = lax.broadcasted_iota(jnp.int32, (NN, KK), 1)
    acc = jnp.zeros((NN, KK), jnp.int32)
    for k in range(KK):
        m = jnp.min(d2, axis=1, keepdims=True)
        am = jnp.min(jnp.where(d2 == m, iota, NN), axis=1, keepdims=True)
        acc = jnp.where(kio == k, am, acc)
        d2 = jnp.where(iota == am, jnp.inf, d2)
    idx_ref[0] = acc + b * NN


def _topk(x):
    return pl.pallas_call(
        _topk_body,
        grid=(BB,),
        in_specs=[pl.BlockSpec((1, NN, 3), lambda b: (b, 0, 0))],
        out_specs=pl.BlockSpec((1, NN, KK), lambda b: (b, 0, 0)),
        out_shape=jax.ShapeDtypeStruct((BB, NN, KK), jnp.int32),
    )(x)


def _pq_body(f_ref, w_ref, p_ref, q_ref):
    f = f_ref[...]
    D = f.shape[1]
    C = w_ref.shape[1]
    CP = p_ref.shape[1]
    wt = w_ref[0:D]
    wb = w_ref[D:]
    if CP != C:
        wt_p = jnp.concatenate(
            [wt, jnp.zeros((D, CP - C), jnp.float32)], axis=1)
    else:
        wt_p = wt
    p_ref[...] = lax.dot_general(f, wt_p, (((1,), (0,)), ((), ())),
                                 preferred_element_type=jnp.float32,
                                 precision=lax.Precision.HIGHEST)
    q_ref[...] = lax.dot_general(f, wb - wt, (((1,), (0,)), ((), ())),
                                 preferred_element_type=jnp.float32,
                                 precision=lax.Precision.HIGHEST)


def _pq(feat, W):
    C = W.shape[1]
    CP = max(C, 128)
    return pl.pallas_call(
        _pq_body,
        out_shape=(jax.ShapeDtypeStruct((BB * NN, CP), jnp.float32),
                   jax.ShapeDtypeStruct((BB * NN, C), jnp.float32)),
    )(feat, W)


def _make_gather_reduce(C, CP):
    GPW = (BB * NN) // NWORK
    CH = 4
    NCH = GPW // CH
    IDXW = GPW * KK
    BN = BB * NN
    mesh = plsc.VectorSubcoreMesh(core_axis_name="c", subcore_axis_name="s")

    @functools.partial(
        pl.kernel,
        mesh=mesh,
        out_type=jax.ShapeDtypeStruct((3 * BN, C), jnp.float32),
        scratch_types=[
            pltpu.VMEM((IDXW,), jnp.int32),
            pltpu.VMEM((CH * KK, CP), jnp.float32),
            pltpu.VMEM((CH, C), jnp.float32),
            pltpu.VMEM((CH, C), jnp.float32),
            pltpu.VMEM((CH, C), jnp.float32),
            pltpu.SemaphoreType.DMA,
        ],
    )
    def sc_kern(p_hbm, idx_hbm, o_hbm, idx_v, rows_v, m_v, s_v, q_v, sem):
        wid = lax.axis_index("s") * 2 + lax.axis_index("c")
        ibase = wid * IDXW
        pltpu.sync_copy(idx_hbm.at[pl.ds(ibase, IDXW)], idx_v)

        @pl.loop(0, NCH)
        def _chunk(ch):
            off = pl.multiple_of(ch * (CH * KK), 8)
            pltpu.async_copy(p_hbm.at[idx_v.at[pl.ds(off, CH * KK)]],
                             rows_v, sem).wait()

            @pl.loop(0, CH)
            def _group(g):
                base = g * KK
                for cb in range(C // 16):
                    sl = pl.ds(cb * 16, 16)
                    v0 = rows_v[base, sl]

                    def kbody(k, carry):
                        vm, vs, vq = carry
                        v = rows_v[base + k, sl]
                        return (jnp.maximum(vm, v), vs + v, vq + v * v)

                    vm, vs, vq = lax.fori_loop(1, KK, kbody,
                                               (v0, v0, v0 * v0))
                    m_v[g, sl] = vm
                    s_v[g, sl] = vs
                    q_v[g, sl] = vq

            row0 = wid * GPW + ch * CH
            pltpu.sync_copy(m_v, o_hbm.at[pl.ds(row0, CH)])
            pltpu.sync_copy(s_v, o_hbm.at[pl.ds(BN + row0, CH)])
            pltpu.sync_copy(q_v, o_hbm.at[pl.ds(2 * BN + row0, CH)])

    return sc_kern


_GR_CACHE = {}


def _gather_reduce(P, idx_flat, C):
    CP = P.shape[1]
    if (C, CP) not in _GR_CACHE:
        _GR_CACHE[(C, CP)] = _make_gather_reduce(C, CP)
    return _GR_CACHE[(C, CP)](P, idx_flat)


def _combine_body(o_ref, q_ref, g_ref, b_ref, x_ref):
    BN = BB * NN
    M = o_ref[0:BN]
    S = o_ref[BN:2 * BN]
    S2 = o_ref[2 * BN:3 * BN]
    Q = q_ref[...]
    cnt = float(BN * KK)
    sum_y = (jnp.sum(S, axis=0, keepdims=True)
             + KK * jnp.sum(Q, axis=0, keepdims=True))
    sumsq = (jnp.sum(S2, axis=0, keepdims=True)
             + 2.0 * jnp.sum(Q * S, axis=0, keepdims=True)
             + KK * jnp.sum(Q * Q, axis=0, keepdims=True))
    mean = sum_y / cnt
    var = sumsq / cnt - mean * mean
    scale = g_ref[...] * lax.rsqrt(var + 1e-5)
    shift = b_ref[...] - mean * scale
    t = (M + Q) * scale + shift
    x_ref[...] = jnp.where(t >= 0, t, SLOPE * t)


def _combine(O, Q, g, b):
    C = Q.shape[1]
    return pl.pallas_call(
        _combine_body,
        out_shape=jax.ShapeDtypeStruct((BB * NN, C), jnp.float32),
    )(O, Q, g.reshape(1, C), b.reshape(1, C))


def _final_body(x1_ref, x2_ref, x3_ref, x4_ref, w_ref, g_ref, b_ref,
                out_ref, max_s, sum_s, sq_s):
    b = pl.program_id(0)
    x5 = lax.dot_general(x1_ref[...], w_ref[0:64],
                         (((1,), (0,)), ((), ())),
                         preferred_element_type=jnp.float32,
                         precision=lax.Precision.HIGHEST)
    x5 += lax.dot_general(x2_ref[...], w_ref[64:128],
                          (((1,), (0,)), ((), ())),
                          preferred_element_type=jnp.float32,
                          precision=lax.Precision.HIGHEST)
    x5 += lax.dot_general(x3_ref[...], w_ref[128:256],
                          (((1,), (0,)), ((), ())),
                          preferred_element_type=jnp.float32,
                          precision=lax.Precision.HIGHEST)
    x5 += lax.dot_general(x4_ref[...], w_ref[256:512],
                          (((1,), (0,)), ((), ())),
                          preferred_element_type=jnp.float32,
                          precision=lax.Precision.HIGHEST)
    bio = lax.broadcasted_iota(jnp.int32, (BB, LAT), 0)
    mx = jnp.max(x5, axis=0, keepdims=True)
    max_s[...] = jnp.where(bio == b, mx, max_s[...])
    sm = jnp.sum(x5, axis=0, keepdims=True)
    sq = jnp.sum(x5 * x5, axis=0, keepdims=True)

    @pl.when(b == 0)
    def _():
        sum_s[...] = sm
        sq_s[...] = sq

    @pl.when(b > 0)
    def _():
        sum_s[...] += sm
        sq_s[...] += sq

    @pl.when(b == BB - 1)
    def _():
        cnt = float(BB * NN)
        mean = sum_s[...] / cnt
        var = sq_s[...] / cnt - mean * mean
        scale = g_ref[...] * lax.rsqrt(var + 1e-5)
        shift = b_ref[...] - mean * scale
        t = max_s[...] * scale + shift
        out_ref[...] = jnp.where(t >= 0, t, SLOPE * t)


def _final(x1, x2, x3, x4, W5, g5, b5):
    return pl.pallas_call(
        _final_body,
        grid=(BB,),
        in_specs=[
            pl.BlockSpec((NN, 64), lambda b: (b, 0)),
            pl.BlockSpec((NN, 64), lambda b: (b, 0)),
            pl.BlockSpec((NN, 128), lambda b: (b, 0)),
            pl.BlockSpec((NN, 256), lambda b: (b, 0)),
            pl.BlockSpec((512, LAT), lambda b: (0, 0)),
            pl.BlockSpec((1, LAT), lambda b: (0, 0)),
            pl.BlockSpec((1, LAT), lambda b: (0, 0)),
        ],
        out_specs=pl.BlockSpec((BB, LAT), lambda b: (0, 0)),
        out_shape=jax.ShapeDtypeStruct((BB, LAT), jnp.float32),
        scratch_shapes=[
            pltpu.VMEM((BB, LAT), jnp.float32),
            pltpu.VMEM((1, LAT), jnp.float32),
            pltpu.VMEM((1, LAT), jnp.float32),
        ],
    )(x1, x2, x3, x4, W5, g5.reshape(1, LAT), b5.reshape(1, LAT))


def kernel(x, W1, W2, W3, W4, W5, g1, b1, g2, b2, g3, b3, g4, b4, g5, b5):
    idx = _topk(x)
    idx_flat = idx.reshape(BB * NN * KK)

    feat = x.reshape(BB * NN, 3)
    outs = []
    for W, g, b in ((W1, g1, b1), (W2, g2, b2), (W3, g3, b3), (W4, g4, b4)):
        P, Q = _pq(feat, W)
        O = _gather_reduce(P, idx_flat, W.shape[1])
        feat = _combine(O, Q, g, b)
        outs.append(feat)

    return _final(outs[0], outs[1], outs[2], outs[3], W5, g5, b5)


# speedup vs baseline: 19.9808x; 2.0208x over previous
import functools
import jax
import jax.numpy as jnp
from jax import lax
from jax.experimental import pallas as pl
from jax.experimental.pallas import tpu as pltpu
from jax.experimental.pallas import tpu_sc as plsc

BB, NN, KK = 8, 1024, 20
LAT = 1024
SLOPE = 0.2
NWORK = 32


def _topk_body(x_ref, idx_ref, cnt_ref):
    b = pl.program_id(0)
    xb = x_ref[0]
    sq = jnp.sum(xb * xb, axis=1, keepdims=True)
    xx = lax.dot_general(xb, xb, (((1,), (1,)), ((), ())),
                         preferred_element_type=jnp.float32)
    d2 = sq + jnp.transpose(sq) - 2.0 * xx
    d2 = jnp.maximum(d2, 0.0)
    iota = lax.broadcasted_iota(jnp.int32, (NN, NN), 1)
    kio = lax.broadcasted_iota(jnp.int32, (NN, KK), 1)
    acc = jnp.zeros((NN, KK), jnp.int32)
    for k in range(KK):
        m = jnp.min(d2, axis=1, keepdims=True)
        am = jnp.min(jnp.where(d2 == m, iota, NN), axis=1, keepdims=True)
        acc = jnp.where(kio == k, am, acc)
        d2 = jnp.where(iota == am, jnp.inf, d2)
    idx_ref[0] = acc + b * NN
    cnt_ref[0] = jnp.sum(jnp.where(d2 == jnp.inf, 1.0, 0.0),
                         axis=0, keepdims=True)


def _topk(x):
    return pl.pallas_call(
        _topk_body,
        grid=(BB,),
        in_specs=[pl.BlockSpec((1, NN, 3), lambda b: (b, 0, 0))],
        out_specs=(pl.BlockSpec((1, NN, KK), lambda b: (b, 0, 0)),
                   pl.BlockSpec((1, 1, NN), lambda b: (b, 0, 0))),
        out_shape=(jax.ShapeDtypeStruct((BB, NN, KK), jnp.int32),
                   jax.ShapeDtypeStruct((BB, 1, NN), jnp.float32)),
    )(x)


def _pq_body(f_ref, w_ref, cnt_ref, p_ref, q_ref, ssq_ref):
    f = f_ref[...]
    D = f.shape[1]
    C = w_ref.shape[1]
    CP = p_ref.shape[1]
    wt = w_ref[0:D]
    wb = w_ref[D:]
    if CP != C:
        wt_p = jnp.concatenate(
            [wt, jnp.zeros((D, CP - C), jnp.float32)], axis=1)
    else:
        wt_p = wt
    p = lax.dot_general(f, wt_p, (((1,), (0,)), ((), ())),
                        preferred_element_type=jnp.float32,
                        precision=lax.Precision.HIGHEST)
    p_ref[...] = p
    q_ref[...] = lax.dot_general(f, wb - wt, (((1,), (0,)), ((), ())),
                                 preferred_element_type=jnp.float32,
                                 precision=lax.Precision.HIGHEST)
    ssq_ref[...] = jnp.sum(p * p * cnt_ref[...], axis=0, keepdims=True)


def _pq(feat, W, cnt):
    C = W.shape[1]
    CP = max(C, 128)
    return pl.pallas_call(
        _pq_body,
        out_shape=(jax.ShapeDtypeStruct((BB * NN, CP), jnp.float32),
                   jax.ShapeDtypeStruct((BB * NN, C), jnp.float32),
                   jax.ShapeDtypeStruct((1, CP), jnp.float32)),
    )(feat, W, cnt)


def _make_gather_reduce(C, CP):
    GPW = (BB * NN) // NWORK
    CH = 4
    NCH = GPW // CH
    IDXW = GPW * KK
    BN = BB * NN
    mesh = plsc.VectorSubcoreMesh(core_axis_name="c", subcore_axis_name="s")

    FCH = 8
    SROW = FCH * CH

    @functools.partial(
        pl.kernel,
        mesh=mesh,
        out_type=jax.ShapeDtypeStruct((2 * BN, C), jnp.float32),
        scratch_types=[
            pltpu.VMEM((IDXW,), jnp.int32),
            pltpu.VMEM((CH * KK, CP), jnp.float32),
            pltpu.VMEM((CH * KK, CP), jnp.float32),
            pltpu.VMEM((SROW, C), jnp.float32),
            pltpu.VMEM((SROW, C), jnp.float32),
            pltpu.SemaphoreType.DMA,
            pltpu.SemaphoreType.DMA,
        ],
    )
    def sc_kern(p_hbm, idx_hbm, o_hbm, idx_v, rows0, rows1, m_st, s_st,
                sem0, sem1):
        wid = lax.axis_index("s") * 2 + lax.axis_index("c")
        pltpu.sync_copy(idx_hbm.at[pl.ds(wid * IDXW, IDXW)], idx_v)

        def issue(ch, buf, sem):
            off = pl.multiple_of(ch * (CH * KK), 8)
            pltpu.async_copy(p_hbm.at[idx_v.at[pl.ds(off, CH * KK)]],
                             buf, sem)

        def wait(buf, sem):
            pltpu.make_async_copy(p_hbm.at[pl.ds(0, CH * KK)], buf,
                                  sem).wait()

        def reduce_store(ch, buf):
            st0 = (lax.rem(ch, FCH)) * CH

            @pl.loop(0, CH)
            def _group(g):
                base = g * KK
                for cb in range(C // 16):
                    sl = pl.ds(cb * 16, 16)
                    va = buf[base, sl]
                    sa = va
                    vb = buf[base + 1, sl]
                    sb = vb
                    for k in range(2, KK, 2):
                        v = buf[base + k, sl]
                        w = buf[base + k + 1, sl]
                        va = jnp.maximum(va, v)
                        sa = sa + v
                        vb = jnp.maximum(vb, w)
                        sb = sb + w
                    m_st[st0 + g, sl] = jnp.maximum(va, vb)
                    s_st[st0 + g, sl] = sa + sb

        issue(0, rows0, sem0)

        @pl.loop(0, NCH, step=2)
        def _chunk(ch):
            issue(ch + 1, rows1, sem1)
            wait(rows0, sem0)
            reduce_store(ch, rows0)

            @pl.when(ch + 2 < NCH)
            def _():
                issue(ch + 2, rows0, sem0)

            wait(rows1, sem1)
            reduce_store(ch + 1, rows1)

            @pl.when(lax.rem(ch, FCH) == FCH - 2)
            def _flush():
                row0 = wid * GPW + (ch - (FCH - 2)) * CH
                pltpu.sync_copy(m_st, o_hbm.at[pl.ds(row0, SROW)])
                pltpu.sync_copy(s_st, o_hbm.at[pl.ds(BN + row0, SROW)])

    return sc_kern


_GR_CACHE = {}


def _gather_reduce(P, idx_flat, C):
    CP = P.shape[1]
    if (C, CP) not in _GR_CACHE:
        _GR_CACHE[(C, CP)] = _make_gather_reduce(C, CP)
    return _GR_CACHE[(C, CP)](P, idx_flat)


def _combine_body(o_ref, q_ref, ssq_ref, g_ref, b_ref, x_ref):
    BN = BB * NN
    M = o_ref[0:BN]
    S = o_ref[BN:2 * BN]
    Q = q_ref[...]
    cnt = float(BN * KK)
    sum_y = (jnp.sum(S, axis=0, keepdims=True)
             + KK * jnp.sum(Q, axis=0, keepdims=True))
    sumsq = (ssq_ref[...]
             + 2.0 * jnp.sum(Q * S, axis=0, keepdims=True)
             + KK * jnp.sum(Q * Q, axis=0, keepdims=True))
    mean = sum_y / cnt
    var = sumsq / cnt - mean * mean
    scale = g_ref[...] * lax.rsqrt(var + 1e-5)
    shift = b_ref[...] - mean * scale
    t = (M + Q) * scale + shift
    x_ref[...] = jnp.where(t >= 0, t, SLOPE * t)


def _combine(O, Q, ssq2, g, b):
    C = Q.shape[1]
    return pl.pallas_call(
        _combine_body,
        out_shape=jax.ShapeDtypeStruct((BB * NN, C), jnp.float32),
    )(O, Q, ssq2[:, :C], g.reshape(1, C), b.reshape(1, C))


def _final_body(x1_ref, x2_ref, x3_ref, x4_ref, w_ref, g_ref, b_ref,
                out_ref, max_s, sum_s, sq_s):
    b = pl.program_id(0)
    x5 = lax.dot_general(x1_ref[...], w_ref[0:64],
                         (((1,), (0,)), ((), ())),
                         preferred_element_type=jnp.float32,
                         precision=lax.Precision.HIGHEST)
    x5 += lax.dot_general(x2_ref[...], w_ref[64:128],
                          (((1,), (0,)), ((), ())),
                          preferred_element_type=jnp.float32,
                          precision=lax.Precision.HIGHEST)
    x5 += lax.dot_general(x3_ref[...], w_ref[128:256],
                          (((1,), (0,)), ((), ())),
                          preferred_element_type=jnp.float32,
                          precision=lax.Precision.HIGHEST)
    x5 += lax.dot_general(x4_ref[...], w_ref[256:512],
                          (((1,), (0,)), ((), ())),
                          preferred_element_type=jnp.float32,
                          precision=lax.Precision.HIGHEST)
    bio = lax.broadcasted_iota(jnp.int32, (BB, LAT), 0)
    mx = jnp.max(x5, axis=0, keepdims=True)
    max_s[...] = jnp.where(bio == b, mx, max_s[...])
    sm = jnp.sum(x5, axis=0, keepdims=True)
    sq = jnp.sum(x5 * x5, axis=0, keepdims=True)

    @pl.when(b == 0)
    def _():
        sum_s[...] = sm
        sq_s[...] = sq

    @pl.when(b > 0)
    def _():
        sum_s[...] += sm
        sq_s[...] += sq

    @pl.when(b == BB - 1)
    def _():
        cnt = float(BB * NN)
        mean = sum_s[...] / cnt
        var = sq_s[...] / cnt - mean * mean
        scale = g_ref[...] * lax.rsqrt(var + 1e-5)
        shift = b_ref[...] - mean * scale
        t = max_s[...] * scale + shift
        out_ref[...] = jnp.where(t >= 0, t, SLOPE * t)


def _final(x1, x2, x3, x4, W5, g5, b5):
    return pl.pallas_call(
        _final_body,
        grid=(BB,),
        in_specs=[
            pl.BlockSpec((NN, 64), lambda b: (b, 0)),
            pl.BlockSpec((NN, 64), lambda b: (b, 0)),
            pl.BlockSpec((NN, 128), lambda b: (b, 0)),
            pl.BlockSpec((NN, 256), lambda b: (b, 0)),
            pl.BlockSpec((512, LAT), lambda b: (0, 0)),
            pl.BlockSpec((1, LAT), lambda b: (0, 0)),
            pl.BlockSpec((1, LAT), lambda b: (0, 0)),
        ],
        out_specs=pl.BlockSpec((BB, LAT), lambda b: (0, 0)),
        out_shape=jax.ShapeDtypeStruct((BB, LAT), jnp.float32),
        scratch_shapes=[
            pltpu.VMEM((BB, LAT), jnp.float32),
            pltpu.VMEM((1, LAT), jnp.float32),
            pltpu.VMEM((1, LAT), jnp.float32),
        ],
    )(x1, x2, x3, x4, W5, g5.reshape(1, LAT), b5.reshape(1, LAT))


def kernel(x, W1, W2, W3, W4, W5, g1, b1, g2, b2, g3, b3, g4, b4, g5, b5):
    idx, cnt = _topk(x)
    idx_flat = idx.reshape(BB * NN * KK)
    cnt_flat = cnt.reshape(BB * NN, 1)

    feat = x.reshape(BB * NN, 3)
    outs = []
    for W, g, b in ((W1, g1, b1), (W2, g2, b2), (W3, g3, b3), (W4, g4, b4)):
        P, Q, ssq2 = _pq(feat, W, cnt_flat)
        O = _gather_reduce(P, idx_flat, W.shape[1])
        feat = _combine(O, Q, ssq2, g, b)
        outs.append(feat)

    return _final(outs[0], outs[1], outs[2], outs[3], W5, g5, b5)


# speedup vs baseline: 20.3484x; 1.0184x over previous
import functools
import jax
import jax.numpy as jnp
from jax import lax
from jax.experimental import pallas as pl
from jax.experimental.pallas import tpu as pltpu
from jax.experimental.pallas import tpu_sc as plsc

BB, NN, KK = 8, 1024, 20
LAT = 1024
SLOPE = 0.2
NWORK = 32


def _pq_part(feat, W, cnt_col, CP):
    D = feat.shape[1]
    C = W.shape[1]
    wt = W[0:D]
    wb = W[D:]
    if CP != C:
        wt_p = jnp.concatenate(
            [wt, jnp.zeros((D, CP - C), jnp.float32)], axis=1)
    else:
        wt_p = wt
    p = lax.dot_general(feat, wt_p, (((1,), (0,)), ((), ())),
                        preferred_element_type=jnp.float32,
                        precision=lax.Precision.HIGHEST)
    q = lax.dot_general(feat, wb - wt, (((1,), (0,)), ((), ())),
                        preferred_element_type=jnp.float32,
                        precision=lax.Precision.HIGHEST)
    ssq = jnp.sum(p * p * cnt_col, axis=0, keepdims=True)
    return p, q, ssq


def _topk_pq1_body(x_ref, w_ref, idx_ref, cnt_ref, p_ref, q_ref, ssq_ref,
                   ssq_acc):
    b = pl.program_id(0)
    xb = x_ref[0]
    sq = jnp.sum(xb * xb, axis=1, keepdims=True)
    xx = lax.dot_general(xb, xb, (((1,), (1,)), ((), ())),
                         preferred_element_type=jnp.float32)
    d2 = sq + jnp.transpose(sq) - 2.0 * xx
    d2 = jnp.maximum(d2, 0.0)
    iota = lax.broadcasted_iota(jnp.int32, (NN, NN), 1)
    kio = lax.broadcasted_iota(jnp.int32, (NN, KK), 1)
    acc = jnp.zeros((NN, KK), jnp.int32)
    for k in range(KK):
        m = jnp.min(d2, axis=1, keepdims=True)
        am = jnp.min(jnp.where(d2 == m, iota, NN), axis=1, keepdims=True)
        acc = jnp.where(kio == k, am, acc)
        d2 = jnp.where(iota == am, jnp.inf, d2)
    idx_ref[0] = acc + b * NN
    cnt_col = jnp.transpose(
        jnp.sum(jnp.where(d2 == jnp.inf, 1.0, 0.0), axis=0, keepdims=True))
    cnt_ref[...] = cnt_col
    p, q, ssq = _pq_part(xb, w_ref[...], cnt_col, p_ref.shape[1])
    p_ref[...] = p
    q_ref[...] = q

    @pl.when(b == 0)
    def _():
        ssq_acc[...] = ssq

    @pl.when(b > 0)
    def _():
        ssq_acc[...] += ssq

    @pl.when(b == BB - 1)
    def _():
        ssq_ref[...] = ssq_acc[...]


def _topk_pq1(x, W1):
    return pl.pallas_call(
        _topk_pq1_body,
        grid=(BB,),
        in_specs=[pl.BlockSpec((1, NN, 3), lambda b: (b, 0, 0)),
                  pl.BlockSpec((6, 64), lambda b: (0, 0))],
        out_specs=(pl.BlockSpec((1, NN, KK), lambda b: (b, 0, 0)),
                   pl.BlockSpec((NN, 1), lambda b: (b, 0)),
                   pl.BlockSpec((NN, 128), lambda b: (b, 0)),
                   pl.BlockSpec((NN, 64), lambda b: (b, 0)),
                   pl.BlockSpec((1, 128), lambda b: (0, 0))),
        out_shape=(jax.ShapeDtypeStruct((BB, NN, KK), jnp.int32),
                   jax.ShapeDtypeStruct((BB * NN, 1), jnp.float32),
                   jax.ShapeDtypeStruct((BB * NN, 128), jnp.float32),
                   jax.ShapeDtypeStruct((BB * NN, 64), jnp.float32),
                   jax.ShapeDtypeStruct((1, 128), jnp.float32)),
        scratch_shapes=[pltpu.VMEM((1, 128), jnp.float32)],
    )(x, W1)


def _make_gather_reduce(C, CP):
    GPW = (BB * NN) // NWORK
    CH = 4
    NCH = GPW // CH
    IDXW = GPW * KK
    BN = BB * NN
    mesh = plsc.VectorSubcoreMesh(core_axis_name="c", subcore_axis_name="s")

    FCH = 8
    SROW = FCH * CH

    @functools.partial(
        pl.kernel,
        mesh=mesh,
        out_type=jax.ShapeDtypeStruct((2 * BN, C), jnp.float32),
        scratch_types=[
            pltpu.VMEM((IDXW,), jnp.int32),
            pltpu.VMEM((CH * KK, CP), jnp.float32),
            pltpu.VMEM((CH * KK, CP), jnp.float32),
            pltpu.VMEM((SROW, C), jnp.float32),
            pltpu.VMEM((SROW, C), jnp.float32),
            pltpu.SemaphoreType.DMA,
            pltpu.SemaphoreType.DMA,
        ],
    )
    def sc_kern(p_hbm, idx_hbm, o_hbm, idx_v, rows0, rows1, m_st, s_st,
                sem0, sem1):
        wid = lax.axis_index("s") * 2 + lax.axis_index("c")
        pltpu.sync_copy(idx_hbm.at[pl.ds(wid * IDXW, IDXW)], idx_v)

        def issue(ch, buf, sem):
            off = pl.multiple_of(ch * (CH * KK), 8)
            pltpu.async_copy(p_hbm.at[idx_v.at[pl.ds(off, CH * KK)]],
                             buf, sem)

        def wait(buf, sem):
            pltpu.make_async_copy(p_hbm.at[pl.ds(0, CH * KK)], buf,
                                  sem).wait()

        def reduce_store(ch, buf):
            st0 = (lax.rem(ch, FCH)) * CH

            @pl.loop(0, CH)
            def _group(g):
                base = g * KK
                for cb in range(C // 16):
                    sl = pl.ds(cb * 16, 16)
                    va = buf[base, sl]
                    sa = va
                    vb = buf[base + 1, sl]
                    sb = vb
                    for k in range(2, KK, 2):
                        v = buf[base + k, sl]
                        w = buf[base + k + 1, sl]
                        va = jnp.maximum(va, v)
                        sa = sa + v
                        vb = jnp.maximum(vb, w)
                        sb = sb + w
                    m_st[st0 + g, sl] = jnp.maximum(va, vb)
                    s_st[st0 + g, sl] = sa + sb

        issue(0, rows0, sem0)

        @pl.loop(0, NCH, step=2)
        def _chunk(ch):
            issue(ch + 1, rows1, sem1)
            wait(rows0, sem0)
            reduce_store(ch, rows0)

            @pl.when(ch + 2 < NCH)
            def _():
                issue(ch + 2, rows0, sem0)

            wait(rows1, sem1)
            reduce_store(ch + 1, rows1)

            @pl.when(lax.rem(ch, FCH) == FCH - 2)
            def _flush():
                row0 = wid * GPW + (ch - (FCH - 2)) * CH
                pltpu.sync_copy(m_st, o_hbm.at[pl.ds(row0, SROW)])
                pltpu.sync_copy(s_st, o_hbm.at[pl.ds(BN + row0, SROW)])

    return sc_kern


_GR_CACHE = {}


def _gather_reduce(P, idx_flat, C):
    CP = P.shape[1]
    if (C, CP) not in _GR_CACHE:
        _GR_CACHE[(C, CP)] = _make_gather_reduce(C, CP)
    return _GR_CACHE[(C, CP)](P, idx_flat)


def _combine_part(o_ref, q_ref, ssq_ref, g_ref, b_ref):
    BN = BB * NN
    M = o_ref[0:BN]
    S = o_ref[BN:2 * BN]
    Q = q_ref[...]
    cnt = float(BN * KK)
    sum_y = (jnp.sum(S, axis=0, keepdims=True)
             + KK * jnp.sum(Q, axis=0, keepdims=True))
    sumsq = (ssq_ref[...]
             + 2.0 * jnp.sum(Q * S, axis=0, keepdims=True)
             + KK * jnp.sum(Q * Q, axis=0, keepdims=True))
    mean = sum_y / cnt
    var = sumsq / cnt - mean * mean
    scale = g_ref[...] * lax.rsqrt(var + 1e-5)
    shift = b_ref[...] - mean * scale
    t = (M + Q) * scale + shift
    return jnp.where(t >= 0, t, SLOPE * t)


def _combine_body(o_ref, q_ref, ssq_ref, g_ref, b_ref, x_ref):
    x_ref[...] = _combine_part(o_ref, q_ref, ssq_ref, g_ref, b_ref)


def _combine(O, Q, ssq2, g, b):
    C = Q.shape[1]
    return pl.pallas_call(
        _combine_body,
        out_shape=jax.ShapeDtypeStruct((BB * NN, C), jnp.float32),
    )(O, Q, ssq2[:, :C], g.reshape(1, C), b.reshape(1, C))


def _pq_body(f_ref, w_ref, cnt_ref, p_ref, q_ref, ssq_ref):
    p, q, ssq = _pq_part(f_ref[...], w_ref[...], cnt_ref[...],
                         p_ref.shape[1])
    p_ref[...] = p
    q_ref[...] = q
    ssq_ref[...] = ssq


def _pq(feat, W, cnt):
    C = W.shape[1]
    CP = max(C, 128)
    return pl.pallas_call(
        _pq_body,
        out_shape=(jax.ShapeDtypeStruct((BB * NN, CP), jnp.float32),
                   jax.ShapeDtypeStruct((BB * NN, C), jnp.float32),
                   jax.ShapeDtypeStruct((1, CP), jnp.float32)),
    )(feat, W, cnt)


def _combine_pq_body(o_ref, q_ref, ssq_ref, g_ref, b_ref, w_ref, cnt_ref,
                     x_ref, p_ref, q2_ref, ssq2_ref):
    xout = _combine_part(o_ref, q_ref, ssq_ref, g_ref, b_ref)
    x_ref[...] = xout
    p, q2, ssq2 = _pq_part(xout, w_ref[...], cnt_ref[...], p_ref.shape[1])
    p_ref[...] = p
    q2_ref[...] = q2
    ssq2_ref[...] = ssq2


def _combine_pq(O, Q, ssq2, g, b, Wn, cnt):
    C = Q.shape[1]
    C2 = Wn.shape[1]
    CP2 = max(C2, 128)
    return pl.pallas_call(
        _combine_pq_body,
        out_shape=(jax.ShapeDtypeStruct((BB * NN, C), jnp.float32),
                   jax.ShapeDtypeStruct((BB * NN, CP2), jnp.float32),
                   jax.ShapeDtypeStruct((BB * NN, C2), jnp.float32),
                   jax.ShapeDtypeStruct((1, CP2), jnp.float32)),
    )(O, Q, ssq2[:, :C], g.reshape(1, C), b.reshape(1, C), Wn, cnt)


def _final_body(x1_ref, x2_ref, x3_ref, x4_ref, w_ref, g_ref, b_ref,
                out_ref, max_s, sum_s, sq_s):
    b = pl.program_id(0)
    x5 = lax.dot_general(x1_ref[...], w_ref[0:64],
                         (((1,), (0,)), ((), ())),
                         preferred_element_type=jnp.float32,
                         precision=lax.Precision.HIGHEST)
    x5 += lax.dot_general(x2_ref[...], w_ref[64:128],
                          (((1,), (0,)), ((), ())),
                          preferred_element_type=jnp.float32,
                          precision=lax.Precision.HIGHEST)
    x5 += lax.dot_general(x3_ref[...], w_ref[128:256],
                          (((1,), (0,)), ((), ())),
                          preferred_element_type=jnp.float32,
                          precision=lax.Precision.HIGHEST)
    x5 += lax.dot_general(x4_ref[...], w_ref[256:512],
                          (((1,), (0,)), ((), ())),
                          preferred_element_type=jnp.float32,
                          precision=lax.Precision.HIGHEST)
    bio = lax.broadcasted_iota(jnp.int32, (BB, LAT), 0)
    mx = jnp.max(x5, axis=0, keepdims=True)
    max_s[...] = jnp.where(bio == b, mx, max_s[...])
    sm = jnp.sum(x5, axis=0, keepdims=True)
    sq = jnp.sum(x5 * x5, axis=0, keepdims=True)

    @pl.when(b == 0)
    def _():
        sum_s[...] = sm
        sq_s[...] = sq

    @pl.when(b > 0)
    def _():
        sum_s[...] += sm
        sq_s[...] += sq

    @pl.when(b == BB - 1)
    def _():
        cnt = float(BB * NN)
        mean = sum_s[...] / cnt
        var = sq_s[...] / cnt - mean * mean
        scale = g_ref[...] * lax.rsqrt(var + 1e-5)
        shift = b_ref[...] - mean * scale
        t = max_s[...] * scale + shift
        out_ref[...] = jnp.where(t >= 0, t, SLOPE * t)


def _final(x1, x2, x3, x4, W5, g5, b5):
    return pl.pallas_call(
        _final_body,
        grid=(BB,),
        in_specs=[
            pl.BlockSpec((NN, 64), lambda b: (b, 0)),
            pl.BlockSpec((NN, 64), lambda b: (b, 0)),
            pl.BlockSpec((NN, 128), lambda b: (b, 0)),
            pl.BlockSpec((NN, 256), lambda b: (b, 0)),
            pl.BlockSpec((512, LAT), lambda b: (0, 0)),
            pl.BlockSpec((1, LAT), lambda b: (0, 0)),
            pl.BlockSpec((1, LAT), lambda b: (0, 0)),
        ],
        out_specs=pl.BlockSpec((BB, LAT), lambda b: (0, 0)),
        out_shape=jax.ShapeDtypeStruct((BB, LAT), jnp.float32),
        scratch_shapes=[
            pltpu.VMEM((BB, LAT), jnp.float32),
            pltpu.VMEM((1, LAT), jnp.float32),
            pltpu.VMEM((1, LAT), jnp.float32),
        ],
    )(x1, x2, x3, x4, W5, g5.reshape(1, LAT), b5.reshape(1, LAT))


def kernel(x, W1, W2, W3, W4, W5, g1, b1, g2, b2, g3, b3, g4, b4, g5, b5):
    idx, cnt_col, P, Q, ssq2 = _topk_pq1(x, W1)
    idx_flat = idx.reshape(BB * NN * KK)

    outs = []
    for W, Wn, g, b in ((W1, W2, g1, b1), (W2, W3, g2, b2),
                        (W3, W4, g3, b3), (W4, None, g4, b4)):
        O = _gather_reduce(P, idx_flat, W.shape[1])
        if Wn is None:
            outs.append(_combine(O, Q, ssq2, g, b))
        elif Wn.shape[1] > 128:
            xi = _combine(O, Q, ssq2, g, b)
            outs.append(xi)
            P, Q, ssq2 = _pq(xi, Wn, cnt_col)
        else:
            xi, P, Q, ssq2 = _combine_pq(O, Q, ssq2, g, b, Wn, cnt_col)
            outs.append(xi)

    return _final(outs[0], outs[1], outs[2], outs[3], W5, g5, b5)
